# baseline (speedup 1.0000x reference)
import functools

import jax
import jax.numpy as jnp
from jax import lax
from jax.experimental import pallas as pl
from jax.experimental.pallas import tpu as pltpu
from jax.experimental.pallas import tpu_sc as plsc

_B = 16384
_T = 10
_F = 5
_V = 20
_NC = 2
_NS = 16
_NW = _NC * _NS
_BPW = _B // _NW
_L = 16
_CHUNKS = _BPW // _L


def _sc_body(idx_hbm, tab_hbm, out_hbm, idx_v, tab_v, t2_v, out_v,
             sem_t, sem_a, sem_b, sem_o):
    wid = lax.axis_index("s") * _NC + lax.axis_index("c")
    base = wid * _BPW
    half = _BPW // 2
    ctab = pltpu.async_copy(tab_hbm, tab_v, sem_t)
    cid_a = pltpu.async_copy(idx_hbm.at[:, pl.ds(base, half)],
                             idx_v.at[:, pl.ds(0, half)], sem_a)
    cid_b = pltpu.async_copy(idx_hbm.at[:, pl.ds(base + half, half)],
                             idx_v.at[:, pl.ds(half, half)], sem_b)

    fsplat = [jnp.full((_L,), f, jnp.int32) for f in range(_F)]
    f400 = [jnp.full((_L,), f * _V * _V, jnp.int32) for f in range(_F)]
    lane = lax.iota(jnp.int32, _L)
    lane4 = lane + 4

    ctab.wait()

    gb0 = [plsc.load_gather(tab_v, [fsplat[f], lane]) for f in range(_F)]
    gb4 = [plsc.load_gather(tab_v, [fsplat[f], lane4]) for f in range(_F)]

    def build(a, carry):
        asplat = jnp.full((_L,), a, jnp.int32)
        for f in range(_F):
            sa = plsc.load_gather(tab_v, [fsplat[f], asplat])
            t2_v[pl.ds(f * _V * _V + a * _V, _L)] = sa * gb0[f]
            t2_v[pl.ds(f * _V * _V + a * _V + 4, _L)] = sa * gb4[f]
        return carry

    lax.fori_loop(0, _V, build, 0)

    def make_half(lo, hi):
        @plsc.parallel_loop(lo, hi, step=1, unroll=2)
        def chunk(c):
            col = c * _L
            i = [idx_v[t, pl.ds(col, _L)] for t in range(_T)]
            p01 = i[0] * _V + i[1]
            p23 = i[2] * _V + i[3]
            p56 = i[5] * _V + i[6]
            p78 = i[7] * _V + i[8]
            for f in range(_F):
                a0 = plsc.load_gather(t2_v, [f400[f] + p01])
                b0 = plsc.load_gather(t2_v, [f400[f] + p23])
                c0 = plsc.load_gather(tab_v, [fsplat[f], i[4]])
                a1 = plsc.load_gather(t2_v, [f400[f] + p56])
                b1 = plsc.load_gather(t2_v, [f400[f] + p78])
                c1 = plsc.load_gather(tab_v, [fsplat[f], i[9]])
                out_v[f, pl.ds(col, _L)] = (a0 * b0 * c0 + a1 * b1 * c1) * 0.5

    cid_a.wait()
    make_half(0, _CHUNKS // 2)
    cout_a = pltpu.async_copy(out_v.at[:, pl.ds(0, half)],
                              out_hbm.at[:, pl.ds(base, half)], sem_o)
    cid_b.wait()
    make_half(_CHUNKS // 2, _CHUNKS)
    cout_a.wait()
    pltpu.sync_copy(out_v.at[:, pl.ds(half, half)],
                    out_hbm.at[:, pl.ds(base + half, half)])


_sc_kernel = functools.partial(
    pl.kernel,
    out_type=jax.ShapeDtypeStruct((_F, _B), jnp.float32),
    mesh=plsc.VectorSubcoreMesh(core_axis_name="c", subcore_axis_name="s"),
    compiler_params=pltpu.CompilerParams(needs_layout_passes=False),
    scratch_types=[
        pltpu.VMEM((_T, _BPW), jnp.int32),
        pltpu.VMEM((_F, _V), jnp.float32),
        pltpu.VMEM((_F * _V * _V,), jnp.float32),
        pltpu.VMEM((_F, _BPW), jnp.float32),
        pltpu.SemaphoreType.DMA,
        pltpu.SemaphoreType.DMA,
        pltpu.SemaphoreType.DMA,
        pltpu.SemaphoreType.DMA,
    ],
)(_sc_body)


def kernel(inputs, table):
    out_t = _sc_kernel(inputs.T.astype(jnp.int32), table.T)
    return out_t.T


# speedup vs baseline: 1.0048x; 1.0048x over previous
import functools

import jax
import jax.numpy as jnp
from jax import lax
from jax.experimental import pallas as pl
from jax.experimental.pallas import tpu as pltpu
from jax.experimental.pallas import tpu_sc as plsc

_B = 16384
_T = 10
_F = 5
_V = 20
_NC = 2
_NS = 16
_NW = _NC * _NS
_BPW = _B // _NW
_L = 16
_CHUNKS = _BPW // _L


def _sc_body(idx_hbm, tab_hbm, out_hbm, idx_v, tab_v, rep_v, out_v,
             sem_t, sem_a, sem_b, sem_o):
    wid = lax.axis_index("s") * _NC + lax.axis_index("c")
    base = wid * _BPW
    half = _BPW // 2
    ctab = pltpu.async_copy(tab_hbm, tab_v, sem_t)
    cid_a = pltpu.async_copy(idx_hbm.at[:, pl.ds(base, half)],
                             idx_v.at[:, pl.ds(0, half)], sem_a)
    cid_b = pltpu.async_copy(idx_hbm.at[:, pl.ds(base + half, half)],
                             idx_v.at[:, pl.ds(half, half)], sem_b)

    fsplat = [jnp.full((_L,), f, jnp.int32) for f in range(_F)]
    lane = lax.iota(jnp.int32, _L)
    cfl = [lax.iota(jnp.int32, _L) + f * _V * _L for f in range(_F)]

    ctab.wait()
    for f in range(_F):
        for r in range(_V):
            vs = plsc.load_gather(tab_v, [fsplat[f], jnp.full((_L,), r, jnp.int32)])
            rep_v[pl.ds((f * _V + r) * _L, _L)] = vs

    def make_half(lo, hi):
        @plsc.parallel_loop(lo, hi, step=1, unroll=2)
        def chunk(c):
            col = c * _L
            acc0 = [None] * _F
            acc1 = [None] * _F
            for t in range(_T):
                s16 = idx_v[t, pl.ds(col, _L)] * _L
                for f in range(_F):
                    v = plsc.load_gather(rep_v, [s16 + cfl[f]])
                    if t < 5:
                        acc0[f] = v if acc0[f] is None else acc0[f] * v
                    else:
                        acc1[f] = v if acc1[f] is None else acc1[f] * v
            for f in range(_F):
                out_v[f, pl.ds(col, _L)] = (acc0[f] + acc1[f]) * 0.5

    cid_a.wait()
    make_half(0, _CHUNKS // 2)
    cout_a = pltpu.async_copy(out_v.at[:, pl.ds(0, half)],
                              out_hbm.at[:, pl.ds(base, half)], sem_o)
    cid_b.wait()
    make_half(_CHUNKS // 2, _CHUNKS)
    cout_a.wait()
    pltpu.sync_copy(out_v.at[:, pl.ds(half, half)],
                    out_hbm.at[:, pl.ds(base + half, half)])


_sc_kernel = functools.partial(
    pl.kernel,
    out_type=jax.ShapeDtypeStruct((_F, _B), jnp.float32),
    mesh=plsc.VectorSubcoreMesh(core_axis_name="c", subcore_axis_name="s"),
    compiler_params=pltpu.CompilerParams(needs_layout_passes=False),
    scratch_types=[
        pltpu.VMEM((_T, _BPW), jnp.int32),
        pltpu.VMEM((_F, _V), jnp.float32),
        pltpu.VMEM((_F * _V * _L,), jnp.float32),
        pltpu.VMEM((_F, _BPW), jnp.float32),
        pltpu.SemaphoreType.DMA,
        pltpu.SemaphoreType.DMA,
        pltpu.SemaphoreType.DMA,
        pltpu.SemaphoreType.DMA,
    ],
)(_sc_body)


def kernel(inputs, table):
    out_t = _sc_kernel(inputs.T.astype(jnp.int32), table.T)
    return out_t.T


# speedup vs baseline: 1.0174x; 1.0126x over previous
import functools

import jax
import jax.numpy as jnp
from jax import lax
from jax.experimental import pallas as pl
from jax.experimental.pallas import tpu as pltpu
from jax.experimental.pallas import tpu_sc as plsc

_B = 16384
_T = 10
_F = 5
_V = 20
_NC = 2
_NS = 16
_NW = _NC * _NS
_BPW = _B // _NW
_L = 16
_CHUNKS = _BPW // _L


def _sc_body(idx_hbm, tab_hbm, out_hbm, idx_v, tab_v, out_v,
             sem_t, sem_a, sem_b, sem_o):
    wid = lax.axis_index("s") * _NC + lax.axis_index("c")
    base = wid * _BPW
    half = _BPW // 2
    ctab = pltpu.async_copy(tab_hbm, tab_v, sem_t)
    cid_a = pltpu.async_copy(idx_hbm.at[:, pl.ds(base, half)],
                             idx_v.at[:, pl.ds(0, half)], sem_a)
    cid_b = pltpu.async_copy(idx_hbm.at[:, pl.ds(base + half, half)],
                             idx_v.at[:, pl.ds(half, half)], sem_b)

    fsplat = [jnp.full((_L,), f, jnp.int32) for f in range(_F)]

    def make_half(lo, hi):
        @plsc.parallel_loop(lo, hi, step=1, unroll=4)
        def chunk(c):
            col = c * _L
            acc0 = [None] * _F
            acc1 = [None] * _F
            for t in range(_T):
                idx16 = idx_v[t, pl.ds(col, _L)]
                for f in range(_F):
                    v = plsc.load_gather(tab_v, [fsplat[f], idx16])
                    if t < 5:
                        acc0[f] = v if acc0[f] is None else acc0[f] * v
                    else:
                        acc1[f] = v if acc1[f] is None else acc1[f] * v
            for f in range(_F):
                out_v[f, pl.ds(col, _L)] = (acc0[f] + acc1[f]) * 0.5

    ctab.wait()
    cid_a.wait()
    make_half(0, _CHUNKS // 2)
    cout_a = pltpu.async_copy(out_v.at[:, pl.ds(0, half)],
                              out_hbm.at[:, pl.ds(base, half)], sem_o)
    cid_b.wait()
    make_half(_CHUNKS // 2, _CHUNKS)
    cout_a.wait()
    pltpu.sync_copy(out_v.at[:, pl.ds(half, half)],
                    out_hbm.at[:, pl.ds(base + half, half)])


_sc_kernel = functools.partial(
    pl.kernel,
    out_type=jax.ShapeDtypeStruct((_F, _B), jnp.float32),
    mesh=plsc.VectorSubcoreMesh(core_axis_name="c", subcore_axis_name="s"),
    compiler_params=pltpu.CompilerParams(needs_layout_passes=False),
    scratch_types=[
        pltpu.VMEM((_T, _BPW), jnp.int32),
        pltpu.VMEM((_F, _V), jnp.float32),
        pltpu.VMEM((_F, _BPW), jnp.float32),
        pltpu.SemaphoreType.DMA,
        pltpu.SemaphoreType.DMA,
        pltpu.SemaphoreType.DMA,
        pltpu.SemaphoreType.DMA,
    ],
)(_sc_body)


def kernel(inputs, table):
    out_t = _sc_kernel(inputs.T.astype(jnp.int32), table.T)
    return out_t.T


# speedup vs baseline: 1.0749x; 1.0565x over previous
import functools

import jax
import jax.numpy as jnp
from jax import lax
from jax.experimental import pallas as pl
from jax.experimental.pallas import tpu as pltpu
from jax.experimental.pallas import tpu_sc as plsc

_B = 16384
_T = 10
_F = 5
_V = 20
_NC = 2
_NS = 16
_NW = _NC * _NS
_BPW = _B // _NW
_L = 16
_CHUNKS = _BPW // _L


def _sc_body(idx_hbm, tab_hbm, out_hbm, idx_v, tab_v, out_v,
             sem_t, sem_a, sem_b, sem_o):
    wid = lax.axis_index("s") * _NC + lax.axis_index("c")
    base = wid * _BPW
    half = _BPW // 2
    ctab = pltpu.async_copy(tab_hbm, tab_v, sem_t)
    cid_a = pltpu.async_copy(idx_hbm.at[:, pl.ds(base, half)],
                             idx_v.at[:, pl.ds(0, half)], sem_a)
    cid_b = pltpu.async_copy(idx_hbm.at[:, pl.ds(base + half, half)],
                             idx_v.at[:, pl.ds(half, half)], sem_b)

    fsplat = [jnp.full((_L,), f, jnp.int32) for f in range(_F)]

    def make_half(lo, hi):
        @plsc.parallel_loop(lo, hi, step=1, unroll=1)
        def chunk(c):
            col = c * _L
            acc0 = [None] * _F
            acc1 = [None] * _F
            for t in range(_T):
                idx16 = idx_v[t, pl.ds(col, _L)]
                for f in range(_F):
                    v = plsc.load_gather(tab_v, [fsplat[f], idx16])
                    if t < 5:
                        acc0[f] = v if acc0[f] is None else acc0[f] * v
                    else:
                        acc1[f] = v if acc1[f] is None else acc1[f] * v
            for f in range(_F):
                out_v[f, pl.ds(col, _L)] = (acc0[f] + acc1[f]) * 0.5

    ctab.wait()
    cid_a.wait()
    make_half(0, _CHUNKS // 2)
    cout_a = pltpu.async_copy(out_v.at[:, pl.ds(0, half)],
                              out_hbm.at[:, pl.ds(base, half)], sem_o)
    cid_b.wait()
    make_half(_CHUNKS // 2, _CHUNKS)
    cout_a.wait()
    pltpu.sync_copy(out_v.at[:, pl.ds(half, half)],
                    out_hbm.at[:, pl.ds(base + half, half)])


_sc_kernel = functools.partial(
    pl.kernel,
    out_type=jax.ShapeDtypeStruct((_F, _B), jnp.float32),
    mesh=plsc.VectorSubcoreMesh(core_axis_name="c", subcore_axis_name="s"),
    compiler_params=pltpu.CompilerParams(needs_layout_passes=False),
    scratch_types=[
        pltpu.VMEM((_T, _BPW), jnp.int32),
        pltpu.VMEM((_F, _V), jnp.float32),
        pltpu.VMEM((_F, _BPW), jnp.float32),
        pltpu.SemaphoreType.DMA,
        pltpu.SemaphoreType.DMA,
        pltpu.SemaphoreType.DMA,
        pltpu.SemaphoreType.DMA,
    ],
)(_sc_body)


def kernel(inputs, table):
    out_t = _sc_kernel(inputs.T.astype(jnp.int32), table.T)
    return out_t.T


# speedup vs baseline: 1.0885x; 1.0127x over previous
import functools

import jax
import jax.numpy as jnp
from jax import lax
from jax.experimental import pallas as pl
from jax.experimental.pallas import tpu as pltpu
from jax.experimental.pallas import tpu_sc as plsc

_B = 16384
_T = 10
_F = 5
_V = 20
_NC = 2
_NS = 16
_NW = _NC * _NS
_BPW = _B // _NW
_L = 16
_CHUNKS = _BPW // _L


def _sc_body(idx_hbm, tab_hbm, out_hbm, idx_v, tab_v, out_v,
             sem_t, sem_a):
    wid = lax.axis_index("s") * _NC + lax.axis_index("c")
    base = wid * _BPW
    ctab = pltpu.async_copy(tab_hbm, tab_v, sem_t)
    cidx = pltpu.async_copy(idx_hbm.at[:, pl.ds(base, _BPW)], idx_v, sem_a)

    fsplat = [jnp.full((_L,), f, jnp.int32) for f in range(_F)]

    ctab.wait()
    cidx.wait()

    @plsc.parallel_loop(0, _CHUNKS, step=1, unroll=1)
    def chunk(c):
        col = c * _L
        acc0 = [None] * _F
        acc1 = [None] * _F
        for t in range(_T):
            idx16 = idx_v[t, pl.ds(col, _L)]
            for f in range(_F):
                v = plsc.load_gather(tab_v, [fsplat[f], idx16])
                if t < 5:
                    acc0[f] = v if acc0[f] is None else acc0[f] * v
                else:
                    acc1[f] = v if acc1[f] is None else acc1[f] * v
        for f in range(_F):
            out_v[f, pl.ds(col, _L)] = (acc0[f] + acc1[f]) * 0.5

    pltpu.sync_copy(out_v, out_hbm.at[:, pl.ds(base, _BPW)])


_sc_kernel = functools.partial(
    pl.kernel,
    out_type=jax.ShapeDtypeStruct((_F, _B), jnp.float32),
    mesh=plsc.VectorSubcoreMesh(core_axis_name="c", subcore_axis_name="s"),
    compiler_params=pltpu.CompilerParams(needs_layout_passes=False),
    scratch_types=[
        pltpu.VMEM((_T, _BPW), jnp.int32),
        pltpu.VMEM((_F, _V), jnp.float32),
        pltpu.VMEM((_F, _BPW), jnp.float32),
        pltpu.SemaphoreType.DMA,
        pltpu.SemaphoreType.DMA,
    ],
)(_sc_body)


def kernel(inputs, table):
    out_t = _sc_kernel(inputs.T.astype(jnp.int32), table.T)
    return out_t.T
